# Initial kernel scaffold; baseline (speedup 1.0000x reference)
import jax
import jax.numpy as jnp
from jax.experimental import pallas as pl


def kernel(p_x, p_edge_attr, v_x, v_edge_attr, v_net_attrs, hidden, params, p_edge_index, p_batch, v_edge_index, v_batch):
    raise NotImplementedError("write your pallas kernel here")


# speedup vs baseline: 18.1020x; 18.1020x over previous
import dataclasses
import functools

import jax
import jax.numpy as jnp
from jax import lax
from jax.experimental import pallas as pl
from jax.experimental.pallas import tpu as pltpu
from jax.experimental.pallas import tpu_sc as plsc

F32 = jnp.float32
EMB = 128
TW = 144
ONES_COL = 128
ALPHA = 0.2
THETA = 0.2
NC, NS = 2, 16
NW = NC * NS
K = 80
ZR = 25

_SC_MESH = plsc.VectorSubcoreMesh(core_axis_name="c", subcore_axis_name="s")

_CP = pltpu.CompilerParams(use_tc_tiling_on_sc=False)
if "needs_layout_passes" in pltpu.CompilerParams.__dataclass_fields__:
    _CP = dataclasses.replace(_CP, needs_layout_passes=False)


def _make_edge_kernel(n_nodes, n_edges):
    C = n_edges // (NW * K)
    CB = 5 if C % 5 == 0 else 1
    rpt = n_nodes // NS

    @functools.partial(
        pl.kernel,
        out_type=jax.ShapeDtypeStruct((NC, n_nodes, TW), F32),
        mesh=_SC_MESH,
        scratch_types=[
            pltpu.VMEM((n_nodes,), F32),
            pltpu.VMEM((n_nodes,), F32),
            pltpu.VMEM((CB, K), jnp.int32),
            pltpu.VMEM((CB, K), jnp.int32),
            pltpu.VMEM((CB, K), F32),
            pltpu.VMEM((K,), F32),
            pltpu.VMEM((K, TW), F32),
            pltpu.VMEM((ZR, TW), F32),
            pltpu.VMEM((16,), F32),
            pltpu.VMEM_SHARED((n_nodes, TW), F32),
            pltpu.SemaphoreType.DMA,
        ],
        compiler_params=_CP,
    )
    def edge_kernel(tab_h, asad_h, src_h, dst_h, ea_h, s_h, out_h,
                    as_t, ad_t, srcb, dstb, eab, wv, rows, zbuf, sv,
                    accum, gsem):
        cid = lax.axis_index("c")
        sid = lax.axis_index("s")
        wid = cid * NS + sid
        row0 = sid * rpt

        @pl.loop(0, ZR)
        def _(r):
            for j in range(TW // 16):
                zbuf[r, pl.ds(j * 16, 16)] = jnp.zeros((16,), F32)

        @pl.loop(0, rpt // ZR)
        def _(i):
            pltpu.sync_copy(zbuf, accum.at[pl.ds(row0 + i * ZR, ZR)])

        pltpu.sync_copy(asad_h.at[0], as_t)
        pltpu.sync_copy(asad_h.at[1], ad_t)
        pltpu.sync_copy(s_h, sv)
        plsc.subcore_barrier()

        @pl.loop(0, C // CB)
        def _(blk):
            pltpu.sync_copy(src_h.at[wid, pl.ds(blk * CB, CB)], srcb)
            pltpu.sync_copy(dst_h.at[wid, pl.ds(blk * CB, CB)], dstb)
            pltpu.sync_copy(ea_h.at[wid, pl.ds(blk * CB, CB)], eab)
            for cc in range(CB):
                gather = pltpu.async_copy(tab_h.at[srcb.at[cc]], rows, gsem)
                shift = sv[...]
                for g in range(K // 16):
                    sl = pl.ds(g * 16, 16)
                    si = srcb[cc, sl]
                    di = dstb[cc, sl]
                    lg = (plsc.load_gather(as_t, [si])
                          + plsc.load_gather(ad_t, [di])
                          + eab[cc, sl])
                    lg = jnp.maximum(lg, ALPHA * lg)
                    wv[sl] = jnp.exp(lg - shift)
                gather.wait()

                @pl.loop(0, K // 16)
                def _(g):
                    wg = wv[pl.ds(g * 16, 16)]
                    for i in range(16):
                        w = wg[i]
                        k = g * 16 + i
                        for j in range(TW // 16):
                            sj = pl.ds(j * 16, 16)
                            rows[k, sj] = rows[k, sj] * w

                pltpu.sync_copy(rows, accum.at[dstb.at[cc]], add=True)

        plsc.subcore_barrier()
        pltpu.sync_copy(accum.at[pl.ds(row0, rpt)],
                        out_h.at[cid, pl.ds(row0, rpt)])

    return edge_kernel


def _dot(a, b):
    return jnp.dot(a, b, precision=lax.Precision.HIGHEST)


def _bf3(a):
    a1 = a.astype(jnp.bfloat16).astype(F32)
    r = a - a1
    a2 = r.astype(jnp.bfloat16).astype(F32)
    a3 = (r - a2).astype(jnp.bfloat16).astype(F32)
    return a1, a2, a3


def _ddot(a, b):
    a1, a2, a3 = _bf3(a)
    b1, b2, b3 = _bf3(b)
    d = lambda x, y: jnp.dot(x, y, precision=lax.Precision.DEFAULT)
    acc = d(a3, b1) + d(a2, b2) + d(a1, b3)
    acc = acc + d(a2, b1) + d(a1, b2)
    return acc + d(a1, b1)


def _ddot_nt(a, b):
    a1, a2, a3 = _bf3(a)
    b1, b2, b3 = _bf3(b)
    d = lambda x, y: lax.dot_general(x, y, (((1,), (1,)), ((), ())),
                                     precision=lax.Precision.DEFAULT)
    acc = d(a3, b1) + d(a2, b2) + d(a1, b3)
    acc = acc + d(a2, b1) + d(a1, b2)
    return acc + d(a1, b1)


def _pc(body, out_shapes, *args):
    return pl.pallas_call(body, out_shape=out_shapes)(*args)


def _k_tab(h):
    n = h.shape[0]

    def body(h_r, tab_r):
        tab_r[...] = jnp.concatenate(
            [h_r[...], jnp.ones((n, 1), F32), jnp.zeros((n, TW - EMB - 1), F32)],
            axis=1)

    return _pc(body, jax.ShapeDtypeStruct((n, TW), F32), h)


def _k_ea(eaT, a_stack):
    e = eaT.shape[1]

    def body(e_r, a_r, out_r, mx_r):
        acc = a_r[:, 0:1] * e_r[0:1, :]
        for j in range(1, 4):
            acc = acc + a_r[:, j:j + 1] * e_r[j:j + 1, :]
        out_r[...] = acc
        mx_r[...] = jnp.max(acc, axis=1, keepdims=True)

    return _pc(body,
               [jax.ShapeDtypeStruct((8, e), F32),
                jax.ShapeDtypeStruct((8, 1), F32)],
               eaT, a_stack)


def _k_pre(parts, h0):
    n = h0.shape[0]

    def body(p_r, h0_r, pre_r):
        acc = p_r[0] + p_r[1]
        num = acc[:, :EMB]
        den = acc[:, ONES_COL:ONES_COL + 1]
        agg = num / (den + 1e-16)
        pre_r[...] = (1.0 - THETA) * agg + THETA * h0_r[...]

    return _pc(body, jax.ShapeDtypeStruct((n, EMB), F32), parts, h0)


def _k_pool_sums(h, bT):
    n = h.shape[0]

    def body(h_r, bT_r, sums_r, cnt_r):
        obT = (lax.broadcasted_iota(jnp.int32, (16, n), 0)
               == bT_r[...]).astype(F32)
        sums_r[...] = _dot(obT, h_r[...])
        cnt_r[...] = jnp.sum(obT, axis=1, keepdims=True)

    return _pc(body,
               [jax.ShapeDtypeStruct((16, EMB), F32),
                jax.ShapeDtypeStruct((16, 1), F32)],
               h, bT)


def _k_score_arg(h, b, ctx):
    n = h.shape[0]

    def body(h_r, b_r, ctx_r, out_r):
        ob = (b_r[...]
              == lax.broadcasted_iota(jnp.int32, (n, 16), 1)).astype(F32)
        out_r[...] = jnp.sum(h_r[...] * _dot(ob, ctx_r[...]), axis=-1,
                             keepdims=True)

    return _pc(body, jax.ShapeDtypeStruct((n, 1), F32), h, b, ctx)


def _k_pooled(h, bT, score):
    n = h.shape[0]

    def body(h_r, bT_r, sc_r, out_r):
        obT = (lax.broadcasted_iota(jnp.int32, (16, n), 0)
               == bT_r[...]).astype(F32)
        out_r[...] = _dot(obT, h_r[...] * sc_r[...])

    return _pc(body, jax.ShapeDtypeStruct((16, EMB), F32), h, bT, score)


def _gat(x, edge_index, edge_attr, p, n_nodes, n_edges):
    h0 = jax.nn.relu(x @ p['W_in'] + p['b_in'])
    a_stack = jnp.concatenate(
        [jnp.stack([lp['a_edge'] for lp in p['layers']]),
         jnp.zeros((3, 4), F32)], axis=0)
    ea_all, eamax = _k_ea(edge_attr.T, a_stack)

    C = n_edges // (NW * K)
    src3 = edge_index[0].reshape(NW, C, K)
    dst3 = edge_index[1].reshape(NW, C, K)
    edge_kernel = _make_edge_kernel(n_nodes, n_edges)

    h = h0
    for l, lp in enumerate(p['layers']):
        a_s = h @ lp['a_src']
        a_d = h @ lp['a_dst']
        m = jnp.max(a_s) + jnp.max(a_d) + eamax[l, 0]
        shift = jnp.broadcast_to(jnp.maximum(m, ALPHA * m), (16,)).astype(F32)
        tab = _k_tab(h)
        asad2 = jnp.stack([a_s, a_d])
        ea3 = ea_all[l].reshape(NW, C, K)
        parts = edge_kernel(tab, asad2, src3, dst3, ea3, shift)
        pre = _k_pre(parts, h0)
        h = jax.nn.relu(pre @ lp['W'] + lp['b'])
    return h


def _mlp(x, layers):
    for i, l in enumerate(layers):
        x = x @ l['W'] + l['b']
        if i < len(layers) - 1:
            x = jax.nn.relu(x)
    return x


def kernel(p_x, p_edge_attr, v_x, v_edge_attr, v_net_attrs, hidden, params,
           p_edge_index, p_batch, v_edge_index, v_batch):
    hp = _gat(p_x, p_edge_index, p_edge_attr, params['gnn_p'],
              p_x.shape[0], p_edge_attr.shape[0])
    hv = _gat(v_x, v_edge_index, v_edge_attr, params['gnn_v'],
              v_x.shape[0], v_edge_attr.shape[0])

    def _gap(h, batch, W):
        bT = batch.astype(jnp.int32)[None, :]
        b = batch.astype(jnp.int32)[:, None]
        sums, cnt = _k_pool_sums(h, bT)
        ctx = jnp.tanh((sums / jnp.maximum(cnt, 1.0)) @ W)
        score = jax.nn.sigmoid(_k_score_arg(h, b, ctx))
        return _k_pooled(h, bT, score)

    gp = _gap(hp, p_batch, params['gap_p'])
    gv = _gap(hv, v_batch, params['gap_v'])
    ga = _mlp(v_net_attrs, params['mlp_attrs'])
    fusion = jnp.concatenate([gp, gv, ga], axis=-1)
    fe = _mlp(fusion, params['mlp_fusion'])
    g = params['gru']
    z = jax.nn.sigmoid(fe @ g['Wxz'] + hidden @ g['Whz'] + g['bz'])
    r = jax.nn.sigmoid(fe @ g['Wxr'] + hidden @ g['Whr'] + g['br'])
    n = jnp.tanh(fe @ g['Wxn'] + r * (hidden @ g['Whn']) + g['bn'])
    new_h = (1.0 - z) * n + z * hidden
    logits = _mlp(new_h, params['actor'])
    return (logits, new_h)
